# Initial kernel scaffold; baseline (speedup 1.0000x reference)
import jax
import jax.numpy as jnp
from jax.experimental import pallas as pl


def kernel(x, edge_index, batch, W1, b1, W2, b2, Wl, bl):
    raise NotImplementedError("write your pallas kernel here")


# speedup vs baseline: 14.4866x; 14.4866x over previous
import functools

import jax
import jax.numpy as jnp
from jax import lax
from jax.experimental import pallas as pl
from jax.experimental.pallas import tpu as pltpu
from jax.experimental.pallas import tpu_sc as plsc

N_NODES = 10000
N_EDGES = 320000
F_IN = 128
F_HID = 64
N_CLS = 3
N_GRAPHS = 64

NC = 2
NS = 16
NW = NC * NS
N_PAD = 10240
RPS = N_PAD // NS
EPW = N_EDGES // NW
CHUNK = 80
ITERS = EPW // CHUNK
ROW_BLK = 1024

_mesh = functools.partial(
    plsc.VectorSubcoreMesh, core_axis_name="c", subcore_axis_name="s"
)


def _sc_degree(dst, zeros_row):
    @functools.partial(
        pl.kernel,
        mesh=_mesh(),
        out_type=jax.ShapeDtypeStruct((NC * N_PAD,), jnp.float32),
        scratch_types=[
            pltpu.VMEM((CHUNK,), jnp.int32),
            pltpu.VMEM((CHUNK,), jnp.float32),
            pltpu.VMEM((CHUNK,), jnp.float32),
            pltpu.VMEM_SHARED((N_PAD,), jnp.float32),
        ],
    )
    def k(dst_hbm, ones_hbm, out_hbm, idx_v, ones_v, stage_v, acc_sh):
        c = lax.axis_index("c")
        s = lax.axis_index("s")
        wid = s * NC + c
        pltpu.sync_copy(ones_hbm.at[pl.ds(0, CHUNK)], ones_v)
        pltpu.sync_copy(ones_hbm.at[pl.ds(CHUNK, CHUNK)], stage_v)

        def zbody(j, _):
            pltpu.sync_copy(
                stage_v, acc_sh.at[pl.ds(s * RPS + j * CHUNK, CHUNK)]
            )
            return _

        lax.fori_loop(0, RPS // CHUNK, zbody, 0)
        plsc.subcore_barrier()

        def body(i, _):
            off = wid * EPW + i * CHUNK
            pltpu.sync_copy(dst_hbm.at[pl.ds(off, CHUNK)], idx_v)
            pltpu.sync_copy(ones_v, acc_sh.at[idx_v], add=True)
            return _

        lax.fori_loop(0, ITERS, body, 0)
        plsc.subcore_barrier()

        def obody(j, _):
            off = s * RPS + j * CHUNK
            pltpu.sync_copy(acc_sh.at[pl.ds(off, CHUNK)], stage_v)
            pltpu.sync_copy(stage_v, out_hbm.at[pl.ds(c * N_PAD + off, CHUNK)])
            return _

        lax.fori_loop(0, RPS // CHUNK, obody, 0)

    return k(dst, zeros_row)


def _sc_agg(g, src, dst, zeros_rows):
    @functools.partial(
        pl.kernel,
        mesh=_mesh(),
        compiler_params=pltpu.CompilerParams(use_tc_tiling_on_sc=False),
        out_type=jax.ShapeDtypeStruct((NC * N_PAD, F_HID), jnp.float32),
        scratch_types=[
            pltpu.VMEM((CHUNK,), jnp.int32),
            pltpu.VMEM((CHUNK,), jnp.int32),
            pltpu.VMEM((CHUNK, F_HID), jnp.float32),
            pltpu.VMEM_SHARED((N_PAD, F_HID), jnp.float32),
            pltpu.SemaphoreType.DMA,
        ],
    )
    def k(g_hbm, src_hbm, dst_hbm, z_hbm, out_hbm, src_v, dst_v, rows_v, acc_sh, sem):
        c = lax.axis_index("c")
        s = lax.axis_index("s")
        wid = s * NC + c

        pltpu.sync_copy(z_hbm, rows_v)

        def zbody(j, _):
            pltpu.sync_copy(
                rows_v, acc_sh.at[pl.ds(s * RPS + j * CHUNK, CHUNK)]
            )
            return _

        lax.fori_loop(0, RPS // CHUNK, zbody, 0)
        plsc.subcore_barrier()

        def body(i, _):
            off = wid * EPW + i * CHUNK
            pltpu.sync_copy(src_hbm.at[pl.ds(off, CHUNK)], src_v)
            pltpu.sync_copy(dst_hbm.at[pl.ds(off, CHUNK)], dst_v)
            pltpu.async_copy(g_hbm.at[src_v], rows_v, sem).wait()
            pltpu.sync_copy(rows_v, acc_sh.at[dst_v], add=True)
            return _

        lax.fori_loop(0, ITERS, body, 0)
        plsc.subcore_barrier()

        def obody(j, _):
            off = s * RPS + j * CHUNK
            pltpu.sync_copy(acc_sh.at[pl.ds(off, CHUNK)], rows_v)
            pltpu.sync_copy(rows_v, out_hbm.at[pl.ds(c * N_PAD + off, CHUNK)])
            return _

        lax.fori_loop(0, RPS // CHUNK, obody, 0)

    return k(g, src, dst, zeros_rows)


def _tc1_body(degp_ref, x_ref, w1_ref, dis_ref, g1_ref):
    deg = degp_ref[:, 0:1] + degp_ref[:, 1:2] + 1.0
    dis = lax.rsqrt(deg)
    dis_ref[...] = dis
    z = jnp.dot(x_ref[...], w1_ref[...], preferred_element_type=jnp.float32)
    g1_ref[...] = z * dis


def _tc1(degp, x, w1):
    grid = N_PAD // ROW_BLK
    return pl.pallas_call(
        _tc1_body,
        grid=(grid,),
        in_specs=[
            pl.BlockSpec((ROW_BLK, 2), lambda i: (i, 0)),
            pl.BlockSpec((ROW_BLK, F_IN), lambda i: (i, 0)),
            pl.BlockSpec((F_IN, F_HID), lambda i: (0, 0)),
        ],
        out_specs=[
            pl.BlockSpec((ROW_BLK, 1), lambda i: (i, 0)),
            pl.BlockSpec((ROW_BLK, F_HID), lambda i: (i, 0)),
        ],
        out_shape=[
            jax.ShapeDtypeStruct((N_PAD, 1), jnp.float32),
            jax.ShapeDtypeStruct((N_PAD, F_HID), jnp.float32),
        ],
    )(degp, x, w1)


def _tc2_body(a0_ref, a1_ref, g1_ref, dis_ref, b1_ref, w2_ref, g2_ref):
    h1 = jnp.maximum(
        (a0_ref[...] + a1_ref[...] + g1_ref[...]) * dis_ref[...] + b1_ref[...],
        0.0,
    )
    z2 = jnp.dot(h1, w2_ref[...], preferred_element_type=jnp.float32)
    g2_ref[...] = z2 * dis_ref[...]


def _tc2(a0, a1, g1, dis, b1, w2):
    grid = N_PAD // ROW_BLK
    rb = pl.BlockSpec((ROW_BLK, F_HID), lambda i: (i, 0))
    return pl.pallas_call(
        _tc2_body,
        grid=(grid,),
        in_specs=[
            rb,
            rb,
            rb,
            pl.BlockSpec((ROW_BLK, 1), lambda i: (i, 0)),
            pl.BlockSpec((1, F_HID), lambda i: (0, 0)),
            pl.BlockSpec((F_HID, F_HID), lambda i: (0, 0)),
        ],
        out_specs=rb,
        out_shape=jax.ShapeDtypeStruct((N_PAD, F_HID), jnp.float32),
    )(a0, a1, g1, dis, b1, w2)


def _tc3_body(
    a0_ref, a1_ref, g2_ref, dis_ref, b2_ref, batch_ref, wl_ref, bl_ref,
    out_ref, pooled_ref, cnt_ref,
):
    i = pl.program_id(0)

    @pl.when(i == 0)
    def _():
        pooled_ref[...] = jnp.zeros_like(pooled_ref)
        cnt_ref[...] = jnp.zeros_like(cnt_ref)

    h2 = jnp.maximum(
        (a0_ref[...] + a1_ref[...] + g2_ref[...]) * dis_ref[...] + b2_ref[...],
        0.0,
    )
    ids = batch_ref[...]
    onehot = (
        ids == lax.broadcasted_iota(jnp.int32, (1, N_GRAPHS), 1)
    ).astype(jnp.float32)
    dn = (((0,), (0,)), ((), ()))
    pooled_ref[...] += lax.dot_general(
        onehot, h2, dn, preferred_element_type=jnp.float32
    )
    cnt_ref[...] += lax.dot_general(
        onehot,
        jnp.ones((ROW_BLK, 1), jnp.float32),
        dn,
        preferred_element_type=jnp.float32,
    )

    @pl.when(i == pl.num_programs(0) - 1)
    def _():
        mean = pooled_ref[...] / jnp.maximum(cnt_ref[...], 1.0)
        out_ref[...] = (
            jnp.dot(mean, wl_ref[...], preferred_element_type=jnp.float32)
            + bl_ref[...]
        )


def _tc3(a0, a1, g2, dis, b2, batchp, wl, bl):
    grid = N_PAD // ROW_BLK
    rb = pl.BlockSpec((ROW_BLK, F_HID), lambda i: (i, 0))
    return pl.pallas_call(
        _tc3_body,
        grid=(grid,),
        in_specs=[
            rb,
            rb,
            rb,
            pl.BlockSpec((ROW_BLK, 1), lambda i: (i, 0)),
            pl.BlockSpec((1, F_HID), lambda i: (0, 0)),
            pl.BlockSpec((ROW_BLK, 1), lambda i: (i, 0)),
            pl.BlockSpec((F_HID, N_CLS), lambda i: (0, 0)),
            pl.BlockSpec((1, N_CLS), lambda i: (0, 0)),
        ],
        out_specs=pl.BlockSpec((N_GRAPHS, N_CLS), lambda i: (0, 0)),
        out_shape=jax.ShapeDtypeStruct((N_GRAPHS, N_CLS), jnp.float32),
        scratch_shapes=[
            pltpu.VMEM((N_GRAPHS, N_GRAPHS), jnp.float32),
            pltpu.VMEM((N_GRAPHS, 1), jnp.float32),
        ],
    )(a0, a1, g2, dis, b2, batchp, wl, bl)


def kernel(x, edge_index, batch, W1, b1, W2, b2, Wl, bl):
    src = edge_index[0]
    dst = edge_index[1]

    x_p = jnp.pad(x, ((0, N_PAD - N_NODES), (0, 0)))
    batch_p = jnp.pad(
        batch, (0, N_PAD - N_NODES), constant_values=N_GRAPHS
    ).reshape(N_PAD, 1)

    ones_zeros = jnp.concatenate(
        [jnp.ones((CHUNK,), jnp.float32), jnp.zeros((CHUNK,), jnp.float32)]
    )
    zeros_rows = jnp.zeros((CHUNK, F_HID), jnp.float32)

    deg_flat = _sc_degree(dst, ones_zeros)
    degp = deg_flat.reshape(NC, N_PAD).T

    dis, g1 = _tc1(degp, x_p, W1)

    acc1 = _sc_agg(g1, src, dst, zeros_rows)
    g2 = _tc2(acc1[:N_PAD], acc1[N_PAD:], g1, dis, b1.reshape(1, F_HID), W2)

    acc2 = _sc_agg(g2, src, dst, zeros_rows)
    out = _tc3(
        acc2[:N_PAD],
        acc2[N_PAD:],
        g2,
        dis,
        b2.reshape(1, F_HID),
        batch_p,
        Wl,
        bl.reshape(1, N_CLS),
    )
    return out
